# baseline (speedup 1.0000x reference)
import jax
import jax.numpy as jnp
from jax import lax
from jax.experimental import pallas as pl
from jax.experimental.pallas import tpu as pltpu
from jax.experimental.pallas import tpu_sc as plsc

B = 16384
D = 32
NC = 2
NS = 16
NW = NC * NS
BW = B // NW
GCH = 128
LANES = 16


def _gmf_body(uidx_hbm, iidx_hbm, utab_hbm, itab_hbm, w_hbm, b_hbm, out_hbm,
              uidx_v, iidx_v, urows_v, irows_v, w_v, b_v, out_v, sem):
    wid = lax.axis_index("s") * NC + lax.axis_index("c")
    base = wid * BW

    copies = []
    for j in range(BW // GCH):
        copies.append(pltpu.async_copy(
            uidx_hbm.at[pl.ds(base + j * GCH, GCH)], uidx_v.at[j], sem))
        copies.append(pltpu.async_copy(
            iidx_hbm.at[pl.ds(base + j * GCH, GCH)], iidx_v.at[j], sem))
    pltpu.sync_copy(w_hbm, w_v)
    pltpu.sync_copy(b_hbm, b_v)
    for c in copies:
        c.wait()

    copies = []
    for j in range(BW // GCH):
        copies.append(pltpu.async_copy(utab_hbm.at[uidx_v.at[j]], urows_v.at[j], sem))
        copies.append(pltpu.async_copy(itab_hbm.at[iidx_v.at[j]], irows_v.at[j], sem))
    for c in copies:
        c.wait()

    wk = [w_v[k, :] for k in range(D)]
    bb = b_v[...]

    lane_iota = lax.iota(jnp.int32, LANES)

    def chunk_body(c, carry):
        rows = c * LANES + lane_iota
        ch = lax.shift_right_logical(rows, 7)
        rw = lax.bitwise_and(rows, jnp.int32(GCH - 1))
        acc = jnp.zeros((LANES,), jnp.float32)
        for k in range(D):
            cols = jnp.full((LANES,), k, jnp.int32)
            gu = plsc.load_gather(urows_v, [ch, rw, cols])
            gi = plsc.load_gather(irows_v, [ch, rw, cols])
            acc = acc + gu * gi * wk[k]
        x = acc + bb
        y = 1.0 / (1.0 + jnp.exp(-x))
        out_v[pl.ds(c * LANES, LANES)] = y
        return carry

    lax.fori_loop(0, BW // LANES, chunk_body, 0)

    pltpu.sync_copy(out_v, out_hbm.at[pl.ds(base, BW)])


@jax.jit
def _gmf(user_indices, item_indices, user_table, item_table, W, b):
    mesh = plsc.VectorSubcoreMesh(core_axis_name="c", subcore_axis_name="s",
                                  num_cores=NC, num_subcores=NS)
    flat = pl.kernel(
        _gmf_body,
        out_type=jax.ShapeDtypeStruct((B,), jnp.float32),
        mesh=mesh,
        compiler_params=pltpu.CompilerParams(needs_layout_passes=False,
                                             use_tc_tiling_on_sc=False),
        scratch_types=[
            pltpu.VMEM((BW // GCH, GCH), jnp.int32),
            pltpu.VMEM((BW // GCH, GCH), jnp.int32),
            pltpu.VMEM((BW // GCH, GCH, D), jnp.float32),
            pltpu.VMEM((BW // GCH, GCH, D), jnp.float32),
            pltpu.VMEM((D, LANES), jnp.float32),
            pltpu.VMEM((LANES,), jnp.float32),
            pltpu.VMEM((BW,), jnp.float32),
            pltpu.SemaphoreType.DMA,
        ],
    )(user_indices, item_indices, user_table, item_table,
      jnp.broadcast_to(W.reshape(D, 1), (D, LANES)),
      jnp.broadcast_to(b, (LANES,)))
    return flat.reshape(B, 1)


def kernel(user_indices, item_indices, user_table, item_table, W, b):
    return _gmf(user_indices, item_indices, user_table, item_table, W, b)
